# Initial kernel scaffold; baseline (speedup 1.0000x reference)
import jax
import jax.numpy as jnp
from jax.experimental import pallas as pl


def kernel(x, table):
    raise NotImplementedError("write your pallas kernel here")


# speedup vs baseline: 2.7147x; 2.7147x over previous
import jax
import jax.numpy as jnp
from jax.experimental import pallas as pl


_ROWS_PER_BLOCK = 512


def _copy_block(in_ref, out_ref):
    out_ref[...] = in_ref[...]


def kernel(x, table):
    del x
    max_len, emb_dim = table.shape
    grid = (max_len // _ROWS_PER_BLOCK,)
    out = pl.pallas_call(
        _copy_block,
        grid=grid,
        in_specs=[pl.BlockSpec((_ROWS_PER_BLOCK, emb_dim), lambda i: (i, 0))],
        out_specs=pl.BlockSpec((_ROWS_PER_BLOCK, emb_dim), lambda i: (i, 0)),
        out_shape=jax.ShapeDtypeStruct((max_len, emb_dim), table.dtype),
    )(table)
    return out[None]
